# Initial kernel scaffold; baseline (speedup 1.0000x reference)
import jax
import jax.numpy as jnp
from jax.experimental import pallas as pl


def kernel(query, key):
    raise NotImplementedError("write your pallas kernel here")


# speedup vs baseline: 6.1203x; 6.1203x over previous
import functools

import jax
import jax.numpy as jnp
from jax.experimental import pallas as pl
from jax.experimental.pallas import tpu as pltpu

QK_DIM = 64
TOPK = 16
SCALE = QK_DIM ** (-0.5)
SEQ = 1024
ROW_BLOCK = 256

NEG_INF = float("-inf")


def _topk_kernel(q_ref, k_ref, w_ref, i_ref):
    q = q_ref[0]
    k = k_ref[0]
    logits = jax.lax.dot_general(
        q * SCALE,
        k,
        (((1,), (1,)), ((), ())),
        preferred_element_type=jnp.float32,
    )

    col = jax.lax.broadcasted_iota(jnp.int32, (ROW_BLOCK, SEQ), 1)
    x = logits
    vals = []
    idxs = []
    for _ in range(TOPK):
        m = jnp.max(x, axis=1, keepdims=True)
        hit = x >= m
        cand = jnp.where(hit, col, SEQ)
        idx = jnp.min(cand, axis=1, keepdims=True)
        vals.append(m)
        idxs.append(idx)
        x = jnp.where(col == idx, NEG_INF, x)

    v = jnp.concatenate(vals, axis=1)
    ix = jnp.concatenate(idxs, axis=1)
    e = jnp.exp(v - v[:, :1])
    w = e / jnp.sum(e, axis=1, keepdims=True)
    w_ref[0] = w
    i_ref[0] = ix


@jax.jit
def kernel(query, key):
    n, s, c = query.shape
    grid = (n, s // ROW_BLOCK)
    w, ix = pl.pallas_call(
        _topk_kernel,
        grid=grid,
        in_specs=[
            pl.BlockSpec((1, ROW_BLOCK, c), lambda b, r: (b, r, 0)),
            pl.BlockSpec((1, s, c), lambda b, r: (b, 0, 0)),
        ],
        out_specs=[
            pl.BlockSpec((1, ROW_BLOCK, TOPK), lambda b, r: (b, r, 0)),
            pl.BlockSpec((1, ROW_BLOCK, TOPK), lambda b, r: (b, r, 0)),
        ],
        out_shape=[
            jax.ShapeDtypeStruct((n, s, TOPK), jnp.float32),
            jax.ShapeDtypeStruct((n, s, TOPK), jnp.int32),
        ],
        compiler_params=pltpu.CompilerParams(
            dimension_semantics=("arbitrary", "arbitrary"),
        ),
    )(query, key)
    return w, ix


# speedup vs baseline: 8.2899x; 1.3545x over previous
import functools

import jax
import jax.numpy as jnp
from jax.experimental import pallas as pl
from jax.experimental.pallas import tpu as pltpu

QK_DIM = 64
TOPK = 16
SCALE = QK_DIM ** (-0.5)
SEQ = 1024
ROW_BLOCK = 256

NEG_INF = float("-inf")


def _topk_kernel(q_ref, k_ref, w_ref, i_ref):
    q = q_ref[0]
    k = k_ref[0]
    logits = jax.lax.dot_general(
        q * SCALE,
        k,
        (((1,), (1,)), ((), ())),
        preferred_element_type=jnp.float32,
    )

    half = SEQ // 2
    a = logits[:, :half]
    b = logits[:, half:]
    colh = jax.lax.broadcasted_iota(jnp.int32, (ROW_BLOCK, half), 1).astype(
        jnp.float32
    )
    o = b > a
    z = jnp.where(o, b, a)
    w = jnp.where(o, a, b)
    kw = jnp.where(o, colh + half, colh)
    kl = jnp.where(o, colh, colh + half)

    vals = []
    idxs = []
    for _ in range(TOPK):
        m = jnp.max(z, axis=1, keepdims=True)
        cand = jnp.where(z >= m, kw, float(SEQ))
        idx = jnp.min(cand, axis=1, keepdims=True)
        vals.append(m)
        idxs.append(idx)
        sel = cand == idx
        z = jnp.where(sel, w, z)
        kw = jnp.where(sel, kl, kw)
        w = jnp.where(sel, NEG_INF, w)

    v = jnp.concatenate(vals, axis=1)
    ix = jnp.concatenate(idxs, axis=1).astype(jnp.int32)
    e = jnp.exp(v - v[:, :1])
    w = e / jnp.sum(e, axis=1, keepdims=True)
    w_ref[0] = w
    i_ref[0] = ix


@jax.jit
def kernel(query, key):
    n, s, c = query.shape
    grid = (n, s // ROW_BLOCK)
    w, ix = pl.pallas_call(
        _topk_kernel,
        grid=grid,
        in_specs=[
            pl.BlockSpec((1, ROW_BLOCK, c), lambda b, r: (b, r, 0)),
            pl.BlockSpec((1, s, c), lambda b, r: (b, 0, 0)),
        ],
        out_specs=[
            pl.BlockSpec((1, ROW_BLOCK, TOPK), lambda b, r: (b, r, 0)),
            pl.BlockSpec((1, ROW_BLOCK, TOPK), lambda b, r: (b, r, 0)),
        ],
        out_shape=[
            jax.ShapeDtypeStruct((n, s, TOPK), jnp.float32),
            jax.ShapeDtypeStruct((n, s, TOPK), jnp.int32),
        ],
        compiler_params=pltpu.CompilerParams(
            dimension_semantics=("arbitrary", "arbitrary"),
        ),
    )(query, key)
    return w, ix


# speedup vs baseline: 8.4906x; 1.0242x over previous
import functools

import jax
import jax.numpy as jnp
from jax.experimental import pallas as pl
from jax.experimental.pallas import tpu as pltpu

QK_DIM = 64
TOPK = 16
SCALE = QK_DIM ** (-0.5)
SEQ = 1024
ROW_BLOCK = 256

NEG_INF = float("-inf")


def _topk_kernel(q_ref, k_ref, w_ref, i_ref):
    q = q_ref[0]
    k = k_ref[0]
    logits = jax.lax.dot_general(
        q * SCALE,
        k,
        (((1,), (1,)), ((), ())),
        preferred_element_type=jnp.float32,
    )

    half = SEQ // 2
    a = logits[:, :half]
    b = logits[:, half:]
    colh = jax.lax.broadcasted_iota(jnp.int32, (ROW_BLOCK, half), 1).astype(
        jnp.float32
    )
    o = b > a
    z = jnp.where(o, b, a)
    w = jnp.where(o, a, b)
    RAD = 2048.0
    kp = jnp.where(
        o, (colh + half) + colh / RAD, colh + (colh + half) / RAD
    )

    for t in range(TOPK):
        m = jnp.max(z, axis=1, keepdims=True)
        cand = jnp.where(z >= m, kp, RAD)
        pidx = jnp.min(cand, axis=1, keepdims=True)
        idx = jnp.floor(pidx)
        w_ref[0, :, t : t + 1] = m
        i_ref[0, :, t : t + 1] = idx.astype(jnp.int32)
        sel = cand == pidx
        swapped = (pidx - idx) * RAD + idx / RAD
        z = jnp.where(sel, w, z)
        kp = jnp.where(sel, swapped, kp)
        w = jnp.where(sel, NEG_INF, w)

    v = w_ref[0]
    e = jnp.exp(v - v[:, :1])
    w_ref[0] = e / jnp.sum(e, axis=1, keepdims=True)


@jax.jit
def kernel(query, key):
    n, s, c = query.shape
    grid = (n, s // ROW_BLOCK)
    w, ix = pl.pallas_call(
        _topk_kernel,
        grid=grid,
        in_specs=[
            pl.BlockSpec((1, ROW_BLOCK, c), lambda b, r: (b, r, 0)),
            pl.BlockSpec((1, s, c), lambda b, r: (b, 0, 0)),
        ],
        out_specs=[
            pl.BlockSpec((1, ROW_BLOCK, TOPK), lambda b, r: (b, r, 0)),
            pl.BlockSpec((1, ROW_BLOCK, TOPK), lambda b, r: (b, r, 0)),
        ],
        out_shape=[
            jax.ShapeDtypeStruct((n, s, TOPK), jnp.float32),
            jax.ShapeDtypeStruct((n, s, TOPK), jnp.int32),
        ],
        compiler_params=pltpu.CompilerParams(
            dimension_semantics=("arbitrary", "arbitrary"),
        ),
    )(query, key)
    return w, ix


# speedup vs baseline: 14.2703x; 1.6807x over previous
import dataclasses
import functools

import jax
import jax.numpy as jnp
from jax.experimental import pallas as pl
from jax.experimental.pallas import tpu as pltpu
from jax.experimental.pallas import tpu_sc as plsc

QK_DIM = 64
TOPK = 16
SCALE = QK_DIM ** (-0.5)
SEQ = 1024


def _logits_kernel(q_ref, k_ref, o_ref):
    o_ref[0] = jax.lax.dot_general(
        q_ref[0] * SCALE,
        k_ref[0],
        (((1,), (1,)), ((), ())),
        preferred_element_type=jnp.float32,
    )


def _logits(q, k):
    n = q.shape[0]
    return pl.pallas_call(
        _logits_kernel,
        grid=(n,),
        in_specs=[
            pl.BlockSpec((1, SEQ, QK_DIM), lambda b: (b, 0, 0)),
            pl.BlockSpec((1, SEQ, QK_DIM), lambda b: (b, 0, 0)),
        ],
        out_specs=pl.BlockSpec((1, SEQ, SEQ), lambda b: (b, 0, 0)),
        out_shape=jax.ShapeDtypeStruct((n, SEQ, SEQ), jnp.float32),
    )(q, k)


def _sc_topk(x):
    rows = x.shape[0]
    mesh = plsc.VectorSubcoreMesh(core_axis_name="c", subcore_axis_name="s")

    cp = pltpu.CompilerParams()
    if "needs_layout_passes" in pltpu.CompilerParams.__dataclass_fields__:
        cp = dataclasses.replace(cp, needs_layout_passes=False)

    @pl.kernel(
        out_type=[
            jax.ShapeDtypeStruct((rows, TOPK), jnp.float32),
            jax.ShapeDtypeStruct((rows, TOPK), jnp.int32),
        ],
        mesh=mesh,
        compiler_params=cp,
    )
    def sck(x_hbm, w_hbm, i_hbm):
        def body(x_vmem, w_vmem, i_vmem):
            xr = x_vmem.at[0]
            cur_v = None
            cur_i = None
            for ch in range(SEQ // TOPK):
                v = xr[pl.ds(ch * TOPK, TOPK)]
                ci = jax.lax.iota(jnp.int32, TOPK) + ch * TOPK
                if cur_v is None:
                    cur_v, cur_i = plsc.sort_key_val(v, ci, descending=True)
                else:
                    sv, si = plsc.sort_key_val(v, ci)
                    mv = jnp.maximum(cur_v, sv)
                    mi = jnp.where(cur_v >= sv, cur_i, si)
                    cur_v, cur_i = plsc.sort_key_val(mv, mi, descending=True)
            m = jnp.max(cur_v)
            e = jnp.exp(cur_v - m)
            w_vmem[0, :] = e / jnp.sum(e)
            i_vmem[0, :] = cur_i

        pltpu.emit_pipeline(
            body,
            grid=(rows,),
            in_specs=[pl.BlockSpec((1, SEQ), lambda r: (r, 0))],
            out_specs=[
                pl.BlockSpec((1, TOPK), lambda r: (r, 0)),
                pl.BlockSpec((1, TOPK), lambda r: (r, 0)),
            ],
            core_axis_name=("c", "s"),
            dimension_semantics=(pltpu.PARALLEL,),
        )(x_hbm, w_hbm, i_hbm)

    return sck(x)


@jax.jit
def kernel(query, key):
    n, s, c = query.shape
    logits = _logits(query, key)
    w, ix = _sc_topk(logits.reshape(n * s, s))
    return w.reshape(n, s, TOPK), ix.reshape(n, s, TOPK)


# speedup vs baseline: 19.6705x; 1.3784x over previous
import dataclasses
import functools

import jax
import jax.numpy as jnp
from jax.experimental import pallas as pl
from jax.experimental.pallas import tpu as pltpu
from jax.experimental.pallas import tpu_sc as plsc

QK_DIM = 64
TOPK = 16
SCALE = QK_DIM ** (-0.5)
SEQ = 1024


def _logits_kernel(q_ref, k_ref, o_ref):
    o_ref[0] = jax.lax.dot_general(
        q_ref[0] * SCALE,
        k_ref[0],
        (((1,), (1,)), ((), ())),
        preferred_element_type=jnp.float32,
    )


def _logits(q, k):
    n = q.shape[0]
    return pl.pallas_call(
        _logits_kernel,
        grid=(n,),
        in_specs=[
            pl.BlockSpec((1, SEQ, QK_DIM), lambda b: (b, 0, 0)),
            pl.BlockSpec((1, SEQ, QK_DIM), lambda b: (b, 0, 0)),
        ],
        out_specs=pl.BlockSpec((1, SEQ, SEQ), lambda b: (b, 0, 0)),
        out_shape=jax.ShapeDtypeStruct((n, SEQ, SEQ), jnp.float32),
    )(q, k)


def _sc_topk(x):
    rows = x.shape[0]
    mesh = plsc.VectorSubcoreMesh(core_axis_name="c", subcore_axis_name="s")

    cp = pltpu.CompilerParams()
    if "needs_layout_passes" in pltpu.CompilerParams.__dataclass_fields__:
        cp = dataclasses.replace(cp, needs_layout_passes=False)

    @pl.kernel(
        out_type=[
            jax.ShapeDtypeStruct((rows, TOPK), jnp.float32),
            jax.ShapeDtypeStruct((rows, TOPK), jnp.int32),
        ],
        mesh=mesh,
        compiler_params=cp,
    )
    def sck(x_hbm, w_hbm, i_hbm):
        def body(x_vmem, w_vmem, i_vmem):
            xr = x_vmem.at[0]
            cur_v = None
            cur_i = None
            for ch in range(SEQ // TOPK):
                v = xr[pl.ds(ch * TOPK, TOPK)]
                ci = jax.lax.iota(jnp.int32, TOPK) + ch * TOPK
                if cur_v is None:
                    cur_v, cur_i = plsc.sort_key_val(v, ci, descending=True)
                else:
                    sv, si = plsc.sort_key_val(v, ci)
                    mv = jnp.maximum(cur_v, sv)
                    mi = jnp.where(cur_v >= sv, cur_i, si)
                    cur_v, cur_i = plsc.sort_key_val(mv, mi, descending=True)
            m = jnp.max(cur_v)
            e = jnp.exp(cur_v - m)
            w_vmem[0, :] = e / jnp.sum(e)
            i_vmem[0, :] = cur_i

        pltpu.emit_pipeline(
            body,
            grid=(rows,),
            in_specs=[pl.BlockSpec((1, SEQ), lambda r: (r, 0))],
            out_specs=[
                pl.BlockSpec((1, TOPK), lambda r: (r, 0)),
                pl.BlockSpec((1, TOPK), lambda r: (r, 0)),
            ],
            core_axis_name=("c", "s"),
            dimension_semantics=(pltpu.PARALLEL,),
        )(x_hbm, w_hbm, i_hbm)

    return sck(x)


N_SLICES = 4


@jax.jit
def kernel(query, key):
    n, s, c = query.shape
    step = n // N_SLICES
    ws, ixs = [], []
    for p in range(N_SLICES):
        qp = query[p * step : (p + 1) * step]
        kp = key[p * step : (p + 1) * step]
        logits = _logits(qp, kp)
        w, ix = _sc_topk(logits.reshape(step * s, s))
        ws.append(w.reshape(step, s, TOPK))
        ixs.append(ix.reshape(step, s, TOPK))
    return jnp.concatenate(ws, axis=0), jnp.concatenate(ixs, axis=0)
